# baseline (speedup 1.0000x reference)
import functools

import jax
import jax.numpy as jnp
from jax import lax
from jax.experimental import pallas as pl
from jax.experimental.pallas import tpu as pltpu
from jax.experimental.pallas import tpu_sc as plsc

_TB = 2048


def _add_kernel(x_ref, tab_ref, o_ref):
    o_ref[...] = x_ref[...] + tab_ref[...]


def _kernel_tc(x, pos_table):
    B, T, D = x.shape
    grid = (T // _TB, B)
    return pl.pallas_call(
        _add_kernel,
        grid=grid,
        in_specs=[
            pl.BlockSpec((1, _TB, D), lambda t, b: (b, t, 0)),
            pl.BlockSpec((_TB, D), lambda t, b: (t, 0)),
        ],
        out_specs=pl.BlockSpec((1, _TB, D), lambda t, b: (b, t, 0)),
        out_shape=jax.ShapeDtypeStruct((B, T, D), x.dtype),
    )(x, pos_table[:T])


_C = 32


def _sc_body(B, T, D, T_PER_W, x_hbm, tab_hbm, out_hbm, xbuf, tabbuf):
    cid = lax.axis_index("c")
    sid = lax.axis_index("s")
    nc = lax.psum(1, "c")
    w = sid * nc + cid
    t0 = w * T_PER_W
    n_chunks = T_PER_W // _C
    groups = D // 16

    def chunk_body(i, _):
        t = t0 + i * _C
        pltpu.sync_copy(tab_hbm.at[pl.ds(t, _C)], tabbuf)

        def batch_body(b, _):
            pltpu.sync_copy(x_hbm.at[b, pl.ds(t, _C)], xbuf)

            def row_body(r, _):
                for c in range(groups):
                    s = pl.ds(c * 16, 16)
                    xbuf[r, s] = xbuf[r, s] + tabbuf[r, s]
                return 0

            lax.fori_loop(0, _C, row_body, 0)
            pltpu.sync_copy(xbuf, out_hbm.at[b, pl.ds(t, _C)])
            return 0

        lax.fori_loop(0, B, batch_body, 0)
        return 0

    lax.fori_loop(0, n_chunks, chunk_body, 0)


def _kernel_sc(x, pos_table):
    B, T, D = x.shape
    info = plsc.get_sparse_core_info()
    nw = info.num_cores * info.num_subcores
    T_PER_W = T // nw
    mesh = plsc.VectorSubcoreMesh(core_axis_name="c", subcore_axis_name="s")
    k = pl.kernel(
        functools.partial(_sc_body, B, T, D, T_PER_W),
        out_type=jax.ShapeDtypeStruct((B, T, D), x.dtype),
        mesh=mesh,
        scratch_types=[
            pltpu.VMEM((_C, D), jnp.float32),
            pltpu.VMEM((_C, D), jnp.float32),
        ],
    )
    return k(x, pos_table[:T])


kernel = _kernel_sc


# speedup vs baseline: 1.4455x; 1.4455x over previous
import functools

import jax
import jax.numpy as jnp
from jax import lax
from jax.experimental import pallas as pl
from jax.experimental.pallas import tpu as pltpu
from jax.experimental.pallas import tpu_sc as plsc

_TB = 2048


def _add_kernel(x_ref, tab_ref, o_ref):
    o_ref[...] = x_ref[...] + tab_ref[...]


def _kernel_tc(x, pos_table):
    B, T, D = x.shape
    grid = (T // _TB, B)
    return pl.pallas_call(
        _add_kernel,
        grid=grid,
        in_specs=[
            pl.BlockSpec((1, _TB, D), lambda t, b: (b, t, 0)),
            pl.BlockSpec((_TB, D), lambda t, b: (t, 0)),
        ],
        out_specs=pl.BlockSpec((1, _TB, D), lambda t, b: (b, t, 0)),
        out_shape=jax.ShapeDtypeStruct((B, T, D), x.dtype),
    )(x, pos_table[:T])


_C = 32


def _sc_body(B, T, D, T_PER_W, x_hbm, tab_hbm, out_hbm, xbuf, tabbuf):
    cid = lax.axis_index("c")
    sid = lax.axis_index("s")
    nc = lax.psum(1, "c")
    w = sid * nc + cid
    t0 = w * T_PER_W
    n_chunks = T_PER_W // _C
    groups = D // 16

    def chunk_body(i, _):
        t = t0 + i * _C
        pltpu.sync_copy(tab_hbm.at[pl.ds(t, _C)], tabbuf)

        def batch_body(b, _):
            pltpu.sync_copy(x_hbm.at[b, pl.ds(t, _C)], xbuf)

            def row_body(r, _):
                for c in range(groups):
                    s = pl.ds(c * 16, 16)
                    xbuf[r, s] = xbuf[r, s] + tabbuf[r, s]
                return 0

            lax.fori_loop(0, _C, row_body, 0)
            pltpu.sync_copy(xbuf, out_hbm.at[b, pl.ds(t, _C)])
            return 0

        lax.fori_loop(0, B, batch_body, 0)
        return 0

    lax.fori_loop(0, n_chunks, chunk_body, 0)


def _sc_body2(B, T, D, T_PER_W, x_hbm, tab_hbm, out_hbm,
              xb0, xb1, tb0,
              isem0, isem1, osem0, osem1):
    cid = lax.axis_index("c")
    sid = lax.axis_index("s")
    nc = lax.psum(1, "c")
    w = sid * nc + cid
    t0 = w * T_PER_W
    xb = (xb0, xb1)
    isem = (isem0, isem1)
    osem = (osem0, osem1)
    n_chunks = T_PER_W // _C
    groups = D // 16

    def x_cp(t, b, p):
        return pltpu.make_async_copy(
            x_hbm.at[b, pl.ds(t, _C)], xb[p], isem[p])

    def out_cp(t, b, p):
        return pltpu.make_async_copy(
            xb[p], out_hbm.at[b, pl.ds(t, _C)], osem[p])

    def chunk_body(i, _):
        t = t0 + i * _C
        x_cp(t, 0, 0).start()
        pltpu.sync_copy(tab_hbm.at[pl.ds(t, _C)], tb0)
        for b in range(B):
            p = b % 2
            if b + 1 < B:
                if b >= 1:
                    out_cp(t, b - 1, 1 - p).wait()
                x_cp(t, b + 1, 1 - p).start()
            x_cp(t, b, p).wait()

            def row_body(r, _):
                for c in range(groups):
                    s = pl.ds(c * 16, 16)
                    xb[p][r, s] = xb[p][r, s] + tb0[r, s]
                return 0

            lax.fori_loop(0, _C, row_body, 0)
            out_cp(t, b, p).start()
        out_cp(t, B - 2, (B - 2) % 2).wait()
        out_cp(t, B - 1, (B - 1) % 2).wait()
        return 0

    lax.fori_loop(0, n_chunks, chunk_body, 0)


def _kernel_sc(x, pos_table):
    B, T, D = x.shape
    info = plsc.get_sparse_core_info()
    nw = info.num_cores * info.num_subcores
    T_PER_W = T // nw
    mesh = plsc.VectorSubcoreMesh(core_axis_name="c", subcore_axis_name="s")
    k = pl.kernel(
        functools.partial(_sc_body2, B, T, D, T_PER_W),
        out_type=jax.ShapeDtypeStruct((B, T, D), x.dtype),
        mesh=mesh,
        scratch_types=[
            pltpu.VMEM((_C, D), jnp.float32),
            pltpu.VMEM((_C, D), jnp.float32),
            pltpu.VMEM((_C, D), jnp.float32),
            pltpu.SemaphoreType.DMA,
            pltpu.SemaphoreType.DMA,
            pltpu.SemaphoreType.DMA,
            pltpu.SemaphoreType.DMA,
        ],
    )
    return k(x, pos_table[:T])


kernel = _kernel_sc


# speedup vs baseline: 2.5300x; 1.7503x over previous
import functools

import jax
import jax.numpy as jnp
from jax import lax
from jax.experimental import pallas as pl
from jax.experimental.pallas import tpu as pltpu
from jax.experimental.pallas import tpu_sc as plsc

_TB = 2048


def _add_kernel(x_ref, tab_ref, o_ref):
    o_ref[...] = x_ref[...] + tab_ref[...]


def _kernel_tc(x, pos_table):
    B, T, D = x.shape
    grid = (T // _TB, B)
    return pl.pallas_call(
        _add_kernel,
        grid=grid,
        in_specs=[
            pl.BlockSpec((1, _TB, D), lambda t, b: (b, t, 0)),
            pl.BlockSpec((_TB, D), lambda t, b: (t, 0)),
        ],
        out_specs=pl.BlockSpec((1, _TB, D), lambda t, b: (b, t, 0)),
        out_shape=jax.ShapeDtypeStruct((B, T, D), x.dtype),
    )(x, pos_table[:T])


_TB2 = 1024


def _kernel_tc2(x, pos_table):
    B, T, D = x.shape
    grid = (T // _TB2,)
    return pl.pallas_call(
        _add_kernel,
        grid=grid,
        in_specs=[
            pl.BlockSpec((B, _TB2, D), lambda t: (0, t, 0)),
            pl.BlockSpec((_TB2, D), lambda t: (t, 0)),
        ],
        out_specs=pl.BlockSpec((B, _TB2, D), lambda t: (0, t, 0)),
        out_shape=jax.ShapeDtypeStruct((B, T, D), x.dtype),
    )(x, pos_table[:T])


_C = 32


def _sc_body(B, T, D, T_PER_W, x_hbm, tab_hbm, out_hbm, xbuf, tabbuf):
    cid = lax.axis_index("c")
    sid = lax.axis_index("s")
    nc = lax.psum(1, "c")
    w = sid * nc + cid
    t0 = w * T_PER_W
    n_chunks = T_PER_W // _C
    groups = D // 16

    def chunk_body(i, _):
        t = t0 + i * _C
        pltpu.sync_copy(tab_hbm.at[pl.ds(t, _C)], tabbuf)

        def batch_body(b, _):
            pltpu.sync_copy(x_hbm.at[b, pl.ds(t, _C)], xbuf)

            def row_body(r, _):
                for c in range(groups):
                    s = pl.ds(c * 16, 16)
                    xbuf[r, s] = xbuf[r, s] + tabbuf[r, s]
                return 0

            lax.fori_loop(0, _C, row_body, 0)
            pltpu.sync_copy(xbuf, out_hbm.at[b, pl.ds(t, _C)])
            return 0

        lax.fori_loop(0, B, batch_body, 0)
        return 0

    lax.fori_loop(0, n_chunks, chunk_body, 0)


def _sc_body2(B, T, D, T_PER_W, x_hbm, tab_hbm, out_hbm,
              xb0, xb1, tb0,
              isem0, isem1, osem0, osem1):
    cid = lax.axis_index("c")
    sid = lax.axis_index("s")
    nc = lax.psum(1, "c")
    w = sid * nc + cid
    t0 = w * T_PER_W
    xb = (xb0, xb1)
    isem = (isem0, isem1)
    osem = (osem0, osem1)
    n_chunks = T_PER_W // _C
    groups = D // 16

    def x_cp(t, b, p):
        return pltpu.make_async_copy(
            x_hbm.at[b, pl.ds(t, _C)], xb[p], isem[p])

    def out_cp(t, b, p):
        return pltpu.make_async_copy(
            xb[p], out_hbm.at[b, pl.ds(t, _C)], osem[p])

    def chunk_body(i, _):
        t = t0 + i * _C
        x_cp(t, 0, 0).start()
        pltpu.sync_copy(tab_hbm.at[pl.ds(t, _C)], tb0)
        for b in range(B):
            p = b % 2
            if b + 1 < B:
                if b >= 1:
                    out_cp(t, b - 1, 1 - p).wait()
                x_cp(t, b + 1, 1 - p).start()
            x_cp(t, b, p).wait()

            def row_body(r, _):
                for c in range(groups):
                    s = pl.ds(c * 16, 16)
                    xb[p][r, s] = xb[p][r, s] + tb0[r, s]
                return 0

            lax.fori_loop(0, _C, row_body, 0)
            out_cp(t, b, p).start()
        out_cp(t, B - 2, (B - 2) % 2).wait()
        out_cp(t, B - 1, (B - 1) % 2).wait()
        return 0

    lax.fori_loop(0, n_chunks, chunk_body, 0)


def _kernel_sc(x, pos_table):
    B, T, D = x.shape
    info = plsc.get_sparse_core_info()
    nw = info.num_cores * info.num_subcores
    T_PER_W = T // nw
    mesh = plsc.VectorSubcoreMesh(core_axis_name="c", subcore_axis_name="s")
    k = pl.kernel(
        functools.partial(_sc_body2, B, T, D, T_PER_W),
        out_type=jax.ShapeDtypeStruct((B, T, D), x.dtype),
        mesh=mesh,
        scratch_types=[
            pltpu.VMEM((_C, D), jnp.float32),
            pltpu.VMEM((_C, D), jnp.float32),
            pltpu.VMEM((_C, D), jnp.float32),
            pltpu.SemaphoreType.DMA,
            pltpu.SemaphoreType.DMA,
            pltpu.SemaphoreType.DMA,
            pltpu.SemaphoreType.DMA,
        ],
    )
    return k(x, pos_table[:T])


kernel = _kernel_tc2


# speedup vs baseline: 2.5426x; 1.0050x over previous
import functools

import jax
import jax.numpy as jnp
from jax import lax
from jax.experimental import pallas as pl
from jax.experimental.pallas import tpu as pltpu
from jax.experimental.pallas import tpu_sc as plsc

_TB = 2048


def _add_kernel(x_ref, tab_ref, o_ref):
    o_ref[...] = x_ref[...] + tab_ref[...]


def _kernel_tc(x, pos_table):
    B, T, D = x.shape
    grid = (T // _TB, B)
    return pl.pallas_call(
        _add_kernel,
        grid=grid,
        in_specs=[
            pl.BlockSpec((1, _TB, D), lambda t, b: (b, t, 0)),
            pl.BlockSpec((_TB, D), lambda t, b: (t, 0)),
        ],
        out_specs=pl.BlockSpec((1, _TB, D), lambda t, b: (b, t, 0)),
        out_shape=jax.ShapeDtypeStruct((B, T, D), x.dtype),
    )(x, pos_table[:T])


_TB2 = 512


def _kernel_tc2(x, pos_table):
    B, T, D = x.shape
    grid = (T // _TB2,)
    return pl.pallas_call(
        _add_kernel,
        grid=grid,
        in_specs=[
            pl.BlockSpec((B, _TB2, D), lambda t: (0, t, 0)),
            pl.BlockSpec((_TB2, D), lambda t: (t, 0)),
        ],
        out_specs=pl.BlockSpec((B, _TB2, D), lambda t: (0, t, 0)),
        out_shape=jax.ShapeDtypeStruct((B, T, D), x.dtype),
    )(x, pos_table[:T])


_C = 32


def _sc_body(B, T, D, T_PER_W, x_hbm, tab_hbm, out_hbm, xbuf, tabbuf):
    cid = lax.axis_index("c")
    sid = lax.axis_index("s")
    nc = lax.psum(1, "c")
    w = sid * nc + cid
    t0 = w * T_PER_W
    n_chunks = T_PER_W // _C
    groups = D // 16

    def chunk_body(i, _):
        t = t0 + i * _C
        pltpu.sync_copy(tab_hbm.at[pl.ds(t, _C)], tabbuf)

        def batch_body(b, _):
            pltpu.sync_copy(x_hbm.at[b, pl.ds(t, _C)], xbuf)

            def row_body(r, _):
                for c in range(groups):
                    s = pl.ds(c * 16, 16)
                    xbuf[r, s] = xbuf[r, s] + tabbuf[r, s]
                return 0

            lax.fori_loop(0, _C, row_body, 0)
            pltpu.sync_copy(xbuf, out_hbm.at[b, pl.ds(t, _C)])
            return 0

        lax.fori_loop(0, B, batch_body, 0)
        return 0

    lax.fori_loop(0, n_chunks, chunk_body, 0)


def _sc_body2(B, T, D, T_PER_W, x_hbm, tab_hbm, out_hbm,
              xb0, xb1, tb0,
              isem0, isem1, osem0, osem1):
    cid = lax.axis_index("c")
    sid = lax.axis_index("s")
    nc = lax.psum(1, "c")
    w = sid * nc + cid
    t0 = w * T_PER_W
    xb = (xb0, xb1)
    isem = (isem0, isem1)
    osem = (osem0, osem1)
    n_chunks = T_PER_W // _C
    groups = D // 16

    def x_cp(t, b, p):
        return pltpu.make_async_copy(
            x_hbm.at[b, pl.ds(t, _C)], xb[p], isem[p])

    def out_cp(t, b, p):
        return pltpu.make_async_copy(
            xb[p], out_hbm.at[b, pl.ds(t, _C)], osem[p])

    def chunk_body(i, _):
        t = t0 + i * _C
        x_cp(t, 0, 0).start()
        pltpu.sync_copy(tab_hbm.at[pl.ds(t, _C)], tb0)
        for b in range(B):
            p = b % 2
            if b + 1 < B:
                if b >= 1:
                    out_cp(t, b - 1, 1 - p).wait()
                x_cp(t, b + 1, 1 - p).start()
            x_cp(t, b, p).wait()

            def row_body(r, _):
                for c in range(groups):
                    s = pl.ds(c * 16, 16)
                    xb[p][r, s] = xb[p][r, s] + tb0[r, s]
                return 0

            lax.fori_loop(0, _C, row_body, 0)
            out_cp(t, b, p).start()
        out_cp(t, B - 2, (B - 2) % 2).wait()
        out_cp(t, B - 1, (B - 1) % 2).wait()
        return 0

    lax.fori_loop(0, n_chunks, chunk_body, 0)


def _kernel_sc(x, pos_table):
    B, T, D = x.shape
    info = plsc.get_sparse_core_info()
    nw = info.num_cores * info.num_subcores
    T_PER_W = T // nw
    mesh = plsc.VectorSubcoreMesh(core_axis_name="c", subcore_axis_name="s")
    k = pl.kernel(
        functools.partial(_sc_body2, B, T, D, T_PER_W),
        out_type=jax.ShapeDtypeStruct((B, T, D), x.dtype),
        mesh=mesh,
        scratch_types=[
            pltpu.VMEM((_C, D), jnp.float32),
            pltpu.VMEM((_C, D), jnp.float32),
            pltpu.VMEM((_C, D), jnp.float32),
            pltpu.SemaphoreType.DMA,
            pltpu.SemaphoreType.DMA,
            pltpu.SemaphoreType.DMA,
            pltpu.SemaphoreType.DMA,
        ],
    )
    return k(x, pos_table[:T])


kernel = _kernel_tc2
